# baseline (device time: 25106 ns/iter reference)
import jax
import jax.numpy as jnp
from jax import lax
from jax.experimental import pallas as pl
from jax.experimental.pallas import tpu as pltpu

T = 256
D = 512
V_SHARD = 4096
V_GLOBAL = 8192
K = 4
R = T // K
QSCALE = 32.0


def kernel(x, W):
    def body(x_ref, w_ref, out_ref, send_buf, recv_buf, send_sems, recv_sems):
        my_x = lax.axis_index("x")
        my_y = lax.axis_index("y")
        peer = (1 - my_x, my_y)

        barrier = pltpu.get_barrier_semaphore()
        pl.semaphore_signal(
            barrier, inc=1, device_id=peer, device_id_type=pl.DeviceIdType.MESH
        )
        pl.semaphore_wait(barrier, 1)

        def chunk_rdma(k):
            return pltpu.make_async_remote_copy(
                src_ref=send_buf.at[k],
                dst_ref=recv_buf.at[k],
                send_sem=send_sems.at[k],
                recv_sem=recv_sems.at[k],
                device_id=peer,
                device_id_type=pl.DeviceIdType.MESH,
            )

        lg = jnp.dot(
            x_ref[...].astype(jnp.bfloat16),
            w_ref[...].astype(jnp.bfloat16),
            preferred_element_type=jnp.float32,
        )
        q = jnp.clip(jnp.round(lg * QSCALE), -127.0, 127.0).astype(jnp.int8)
        send_buf[...] = q.reshape(K, R, V_SHARD)
        for k in range(K):
            chunk_rdma(k).start()

        e_loc = jnp.exp(lg)
        s_loc = jnp.sum(e_loc, -1, keepdims=True)

        for k in range(K):
            chunk_rdma(k).wait_recv()
            e_rem = jnp.exp(recv_buf[k].astype(jnp.float32) * (1.0 / QSCALE))
            rows_static = slice(k * R, (k + 1) * R)
            s = s_loc[rows_static] + jnp.sum(e_rem, -1, keepdims=True)
            inv = 1.0 / s
            rows = pl.ds(k * R, R)
            out_ref[rows, pl.ds(my_x * V_SHARD, V_SHARD)] = e_loc[rows_static] * inv
            out_ref[rows, pl.ds((1 - my_x) * V_SHARD, V_SHARD)] = e_rem * inv

        for k in range(K):
            chunk_rdma(k).wait_send()

    return pl.pallas_call(
        body,
        out_shape=jax.ShapeDtypeStruct((T, V_GLOBAL), jnp.float32),
        in_specs=[
            pl.BlockSpec(memory_space=pltpu.VMEM),
            pl.BlockSpec(memory_space=pltpu.VMEM),
        ],
        out_specs=pl.BlockSpec(memory_space=pltpu.VMEM),
        scratch_shapes=[
            pltpu.VMEM((K, R, V_SHARD), jnp.int8),
            pltpu.VMEM((K, R, V_SHARD), jnp.int8),
            pltpu.SemaphoreType.DMA((K,)),
            pltpu.SemaphoreType.DMA((K,)),
        ],
        compiler_params=pltpu.CompilerParams(collective_id=0),
    )(x, W)


# device time: 24612 ns/iter; 1.0201x vs baseline; 1.0201x over previous
import jax
import jax.numpy as jnp
from jax import lax
from jax.experimental import pallas as pl
from jax.experimental.pallas import tpu as pltpu

T = 256
D = 512
V_SHARD = 4096
V_GLOBAL = 8192
H = T // 2
C = 2
R = H // C
QSCALE = 32.0


def kernel(x, W):
    def body(
        x_ref,
        w_ref,
        out_ref,
        qbuf,
        eloc_buf,
        xrecv,
        yrecv,
        x_send_sems,
        x_recv_sems,
        y_send_sems,
        y_recv_sems,
    ):
        my_x = lax.axis_index("x")
        my_y = lax.axis_index("y")
        x_peer = (1 - my_x, my_y)
        y_peer = (my_x, 1 - my_y)

        barrier = pltpu.get_barrier_semaphore()
        for nbr in (x_peer, y_peer):
            pl.semaphore_signal(
                barrier, inc=1, device_id=nbr, device_id_type=pl.DeviceIdType.MESH
            )
        pl.semaphore_wait(barrier, 2)

        def x_rdma(c):
            return pltpu.make_async_remote_copy(
                src_ref=qbuf.at[pl.ds(my_y * H + c * R, R), :],
                dst_ref=xrecv.at[c],
                send_sem=x_send_sems.at[c],
                recv_sem=x_recv_sems.at[c],
                device_id=x_peer,
                device_id_type=pl.DeviceIdType.MESH,
            )

        def y_rdma(c):
            return pltpu.make_async_remote_copy(
                src_ref=xrecv.at[c],
                dst_ref=yrecv.at[c],
                send_sem=y_send_sems.at[c],
                recv_sem=y_recv_sems.at[c],
                device_id=y_peer,
                device_id_type=pl.DeviceIdType.MESH,
            )

        lg = jnp.dot(
            x_ref[...].astype(jnp.bfloat16),
            w_ref[...].astype(jnp.bfloat16),
            preferred_element_type=jnp.float32,
        )
        qbuf[...] = jnp.clip(jnp.round(lg * QSCALE), -127.0, 127.0).astype(jnp.int8)
        for c in range(C):
            x_rdma(c).start()

        eloc_buf[...] = jnp.exp(lg)

        def softmax_rows(row0, e_rem):
            el = eloc_buf[pl.ds(row0, R), :]
            s = jnp.sum(el, -1, keepdims=True) + jnp.sum(e_rem, -1, keepdims=True)
            inv = 1.0 / s
            rows = pl.ds(row0, R)
            out_ref[rows, pl.ds(my_x * V_SHARD, V_SHARD)] = el * inv
            out_ref[rows, pl.ds((1 - my_x) * V_SHARD, V_SHARD)] = e_rem * inv

        for c in range(C):
            x_rdma(c).wait_recv()
            y_rdma(c).start()
            e_rem = jnp.exp(xrecv[c].astype(jnp.float32) * (1.0 / QSCALE))
            softmax_rows(my_y * H + c * R, e_rem)

        for c in range(C):
            y_rdma(c).wait_recv()
            e_rem = jnp.exp(yrecv[c].astype(jnp.float32) * (1.0 / QSCALE))
            softmax_rows((1 - my_y) * H + c * R, e_rem)

        for c in range(C):
            x_rdma(c).wait_send()
            y_rdma(c).wait_send()

    return pl.pallas_call(
        body,
        out_shape=jax.ShapeDtypeStruct((T, V_GLOBAL), jnp.float32),
        in_specs=[
            pl.BlockSpec(memory_space=pltpu.VMEM),
            pl.BlockSpec(memory_space=pltpu.VMEM),
        ],
        out_specs=pl.BlockSpec(memory_space=pltpu.VMEM),
        scratch_shapes=[
            pltpu.VMEM((T, V_SHARD), jnp.int8),
            pltpu.VMEM((T, V_SHARD), jnp.float32),
            pltpu.VMEM((C, R, V_SHARD), jnp.int8),
            pltpu.VMEM((C, R, V_SHARD), jnp.int8),
            pltpu.SemaphoreType.DMA((C,)),
            pltpu.SemaphoreType.DMA((C,)),
            pltpu.SemaphoreType.DMA((C,)),
            pltpu.SemaphoreType.DMA((C,)),
        ],
        compiler_params=pltpu.CompilerParams(collective_id=0),
    )(x, W)


# device time: 23429 ns/iter; 1.0716x vs baseline; 1.0505x over previous
import jax
import jax.numpy as jnp
from jax import lax
from jax.experimental import pallas as pl
from jax.experimental.pallas import tpu as pltpu

T = 256
D = 512
V_SHARD = 4096
V_GLOBAL = 8192
K = 4
R = T // K
QSCALE = 32.0


def kernel(x, W):
    def body(x_ref, w_ref, out_ref, send_buf, recv_buf, send_sems, recv_sems):
        my_x = lax.axis_index("x")
        my_y = lax.axis_index("y")
        peer = (1 - my_x, my_y)

        barrier = pltpu.get_barrier_semaphore()
        pl.semaphore_signal(
            barrier, inc=1, device_id=peer, device_id_type=pl.DeviceIdType.MESH
        )
        pl.semaphore_wait(barrier, 1)

        def chunk_rdma(k):
            return pltpu.make_async_remote_copy(
                src_ref=send_buf.at[k],
                dst_ref=recv_buf.at[k],
                send_sem=send_sems.at[k],
                recv_sem=recv_sems.at[k],
                device_id=peer,
                device_id_type=pl.DeviceIdType.MESH,
            )

        lg = jnp.dot(
            x_ref[...].astype(jnp.bfloat16),
            w_ref[...].astype(jnp.bfloat16),
            preferred_element_type=jnp.float32,
        )
        for k in range(K):
            lgk = lg[k * R : (k + 1) * R]
            send_buf[k] = jnp.clip(jnp.round(lgk * QSCALE), -127.0, 127.0).astype(
                jnp.int8
            )
            chunk_rdma(k).start()

        for k in range(K):
            chunk_rdma(k).wait_recv()
            e_loc = jnp.exp(lg[k * R : (k + 1) * R])
            e_rem = jnp.exp(recv_buf[k].astype(jnp.float32) * (1.0 / QSCALE))
            s = jnp.sum(e_loc, -1, keepdims=True) + jnp.sum(e_rem, -1, keepdims=True)
            inv = 1.0 / s
            rows = pl.ds(k * R, R)
            out_ref[rows, pl.ds(my_x * V_SHARD, V_SHARD)] = (e_loc * inv).astype(
                jnp.bfloat16
            )
            out_ref[rows, pl.ds((1 - my_x) * V_SHARD, V_SHARD)] = (e_rem * inv).astype(
                jnp.bfloat16
            )

        for k in range(K):
            chunk_rdma(k).wait_send()

    return pl.pallas_call(
        body,
        out_shape=jax.ShapeDtypeStruct((T, V_GLOBAL), jnp.bfloat16),
        in_specs=[
            pl.BlockSpec(memory_space=pltpu.VMEM),
            pl.BlockSpec(memory_space=pltpu.VMEM),
        ],
        out_specs=pl.BlockSpec(memory_space=pltpu.VMEM),
        scratch_shapes=[
            pltpu.VMEM((K, R, V_SHARD), jnp.int8),
            pltpu.VMEM((K, R, V_SHARD), jnp.int8),
            pltpu.SemaphoreType.DMA((K,)),
            pltpu.SemaphoreType.DMA((K,)),
        ],
        compiler_params=pltpu.CompilerParams(collective_id=0),
    )(x, W)
